# baseline (speedup 1.0000x reference)
import functools

import jax
import jax.numpy as jnp
from jax import lax
from jax.experimental import pallas as pl
from jax.experimental.pallas import tpu as pltpu
from jax.experimental.pallas import tpu_sc as plsc

_N = 10000
_E = 320000
_G = 64
_DIN = 128
_DHID = 64
_DOUT = 8

_NC = 2
_NS = 16
_CHUNK = 128
_NT = 10240
_RPT = _NT // _NS

_TRIPS = 80
_EPAD = _TRIPS * _CHUNK * _NC * _NS
_NCHUNKS = _EPAD // _CHUNK


def _sc_mesh():
    return plsc.VectorSubcoreMesh(core_axis_name="c", subcore_axis_name="s")


def _deg_kernel(col_hbm, out_hbm, col2d, ones_v, zb, acc, sem):
    c = lax.axis_index("c")
    s = lax.axis_index("s")
    for i in range(_CHUNK // 16):
        ones_v[pl.ds(16 * i, 16)] = jnp.ones((16,), jnp.float32)
        zb[pl.ds(16 * i, 16)] = jnp.zeros((16,), jnp.float32)
    for i in range(_RPT // _CHUNK):
        pltpu.sync_copy(zb, acc.at[pl.ds(s * _RPT + _CHUNK * i, _CHUNK)])
    trip0 = (c * _NS + s) * _TRIPS
    pltpu.sync_copy(col_hbm.at[pl.ds(trip0, _TRIPS)], col2d)
    plsc.subcore_barrier()

    @pl.loop(0, _TRIPS // 16)
    def _(g):
        descs = []
        for b in range(16):
            t = g * 16 + b
            descs.append(pltpu.async_copy(ones_v, acc.at[col2d.at[t]], sem, add=True))
        for d in descs:
            d.wait()

    plsc.subcore_barrier()
    pltpu.sync_copy(acc.at[pl.ds(s * _RPT, _RPT)], out_hbm.at[c, pl.ds(s * _RPT, _RPT)])


def _make_deg():
    return pl.kernel(
        _deg_kernel,
        out_type=jax.ShapeDtypeStruct((_NC, _NT), jnp.float32),
        mesh=_sc_mesh(),
        compiler_params=pltpu.CompilerParams(use_tc_tiling_on_sc=False),
        scratch_types=[
            pltpu.VMEM((_TRIPS, _CHUNK), jnp.int32),
            pltpu.VMEM((_CHUNK,), jnp.float32),
            pltpu.VMEM((_CHUNK,), jnp.float32),
            pltpu.MemorySpace.VMEM_SHARED((_NT,), jnp.float32),
            pltpu.SemaphoreType.DMA,
        ],
    )


def _make_scatter(d, kd):
    groups = _TRIPS // kd
    inline_zero = d % 16 == 0

    def inner(row_hbm, col_hbm, g_hbm, zeros_hbm, out_hbm,
              row2d, col2d, rows, acc, gsem, ssem):
        c = lax.axis_index("c")
        s = lax.axis_index("s")
        if inline_zero:
            r0 = rows.at[0]

            @pl.loop(0, _CHUNK)
            def _(j):
                for i in range(d // 16):
                    r0[j, pl.ds(16 * i, 16)] = jnp.zeros((16,), jnp.float32)

            for i in range(_RPT // _CHUNK):
                pltpu.sync_copy(r0, acc.at[pl.ds(s * _RPT + _CHUNK * i, _CHUNK)])
        else:
            pltpu.sync_copy(zeros_hbm.at[pl.ds(s * _RPT, _RPT)],
                            acc.at[pl.ds(s * _RPT, _RPT)])
        trip0 = (c * _NS + s) * _TRIPS
        pltpu.sync_copy(row_hbm.at[pl.ds(trip0, _TRIPS)], row2d)
        pltpu.sync_copy(col_hbm.at[pl.ds(trip0, _TRIPS)], col2d)
        plsc.subcore_barrier()

        for b in range(kd):
            pltpu.async_copy(g_hbm.at[row2d.at[b]], rows.at[b], gsem)

        @pl.loop(0, groups)
        def _(g):
            t0 = g * kd
            sdescs = []
            for b in range(kd):
                t = t0 + b
                pltpu.make_async_copy(g_hbm.at[row2d.at[t]], rows.at[b], gsem).wait()
                sdescs.append(
                    pltpu.async_copy(rows.at[b], acc.at[col2d.at[t]], ssem, add=True))
            for b in range(kd):
                sdescs[b].wait()
                tn = t0 + kd + b
                tn = jnp.where(tn >= _TRIPS, tn - _TRIPS, tn)
                pltpu.async_copy(g_hbm.at[row2d.at[tn]], rows.at[b], gsem)

        for b in range(kd):
            pltpu.make_async_copy(g_hbm.at[row2d.at[b]], rows.at[b], gsem).wait()
        plsc.subcore_barrier()
        pltpu.sync_copy(acc.at[pl.ds(s * _RPT, _RPT)],
                        out_hbm.at[c, pl.ds(s * _RPT, _RPT)])

    if inline_zero:
        def body(row_hbm, col_hbm, g_hbm, out_hbm,
                 row2d, col2d, rows, acc, gsem, ssem):
            inner(row_hbm, col_hbm, g_hbm, None, out_hbm,
                  row2d, col2d, rows, acc, gsem, ssem)
    else:
        body = inner

    return pl.kernel(
        body,
        out_type=jax.ShapeDtypeStruct((_NC, _NT, d), jnp.float32),
        mesh=_sc_mesh(),
        compiler_params=pltpu.CompilerParams(use_tc_tiling_on_sc=False),
        scratch_types=[
            pltpu.VMEM((_TRIPS, _CHUNK), jnp.int32),
            pltpu.VMEM((_TRIPS, _CHUNK), jnp.int32),
            pltpu.VMEM((kd, _CHUNK, d), jnp.float32),
            pltpu.MemorySpace.VMEM_SHARED((_NT, d), jnp.float32),
            pltpu.SemaphoreType.DMA,
            pltpu.SemaphoreType.DMA,
        ],
    )


def _sel(pairs, width):
    seg = width // pairs
    lane = lax.broadcasted_iota(jnp.int32, (pairs, width), 1)
    row = lax.broadcasted_iota(jnp.int32, (pairs, width), 0)
    return (lane // seg == row).astype(jnp.float32)


def _dis2(degp2_ref):
    d2 = degp2_ref[0] + degp2_ref[1]
    return lax.rsqrt(d2[:_N // 2] + 1.0)


def _mm1_body(xpk_ref, w1blk_ref, degp2_ref, g1p_ref):
    dis_pk = lax.dot_general(_dis2(degp2_ref), _sel(2, 128), (((1,), (0,)), ((), ())),
                             preferred_element_type=jnp.float32)
    hpk = jnp.dot(xpk_ref[...], w1blk_ref[...],
                  preferred_element_type=jnp.float32)
    g1p_ref[...] = hpk * dis_pk


def _mm2_body(s1p_ref, g1p_ref, degp2_ref, b1pk_ref, w2blk_ref, g2p_ref):
    cn = (((1,), (0,)), ((), ()))
    dis2 = _dis2(degp2_ref)
    dis_pk = lax.dot_general(dis2, _sel(2, 128), cn,
                             preferred_element_type=jnp.float32)
    agg = (s1p_ref[0] + s1p_ref[1])[:_N // 2] + g1p_ref[...]
    h1pk = jnp.tanh(dis_pk * agg + b1pk_ref[...])
    z2pk = jnp.dot(h1pk, w2blk_ref[...],
                   preferred_element_type=jnp.float32)
    dis_pk16 = lax.dot_general(dis2, _sel(2, 16), cn,
                               preferred_element_type=jnp.float32)
    g2p_ref[...] = z2pk * dis_pk16


def _tail_body(s2p_ref, g2p16_ref, degp16_ref, b2pk_ref, batchp_ref, out_ref):
    cn = (((1,), (0,)), ((), ()))
    d16 = degp16_ref[0] + degp16_ref[1]
    dis16 = lax.rsqrt(d16[:_N // 16] + 1.0)
    dis_pk = lax.dot_general(dis16, _sel(16, 128), cn,
                             preferred_element_type=jnp.float32)
    agg = (s2p_ref[0] + s2p_ref[1])[:_N // 16] + g2p16_ref[...]
    h2pk = jnp.tanh(dis_pk * agg + b2pk_ref[...])
    batchp = batchp_ref[...]
    gid = lax.broadcasted_iota(jnp.int32, (1, _G), 1)
    ones = jnp.ones((_N // 16, 1), jnp.float32)
    dn0 = (((0,), (0,)), ((), ()))
    sums = jnp.zeros((_G, _DOUT), jnp.float32)
    cnt = jnp.zeros((_G, 1), jnp.float32)
    for k in range(16):
        mk = (batchp[:, k:k + 1] == gid).astype(jnp.float32)
        hk = h2pk[:, 8 * k:8 * k + 8]
        sums = sums + lax.dot_general(mk, hk, dn0,
                                      preferred_element_type=jnp.float32)
        cnt = cnt + lax.dot_general(mk, ones, dn0,
                                    preferred_element_type=jnp.float32)
    mean = sums / jnp.maximum(cnt, 1.0)
    out_ref[...] = 1.0 / (1.0 + jnp.exp(-mean))


def kernel(x, edge_index, batch_index, W1, b1, W2, b2):
    row = edge_index[0].astype(jnp.int32)
    col = edge_index[1].astype(jnp.int32)
    npad = _EPAD - _E
    pad_r = (jnp.arange(npad, dtype=jnp.int32) * 37) % _N
    pad_c = _N + (jnp.arange(npad, dtype=jnp.int32) % (_NT - _N))
    row_p = jnp.concatenate([row, pad_r]).reshape(_NCHUNKS, _CHUNK)
    col_p = jnp.concatenate([col, pad_c]).reshape(_NCHUNKS, _CHUNK)

    zeros8 = jnp.zeros((_NT, _DOUT), jnp.float32)

    degp = _make_deg()(col_p)

    degp2 = degp.reshape(_NC, _NT // 2, 2)
    zpad1 = jnp.zeros((_DIN, _DHID), jnp.float32)
    w1blk = jnp.concatenate(
        [jnp.concatenate([W1, zpad1], axis=1),
         jnp.concatenate([zpad1, W1], axis=1)], axis=0)
    g1p = pl.pallas_call(
        _mm1_body,
        out_shape=jax.ShapeDtypeStruct((_N // 2, 2 * _DHID), jnp.float32),
    )(x.reshape(_N // 2, 2 * _DIN), w1blk, degp2)

    s1 = _make_scatter(_DHID, 8)(
        row_p, col_p, g1p.reshape(_N, _DHID))

    b1pk = jnp.concatenate([b1, b1]).reshape(1, 2 * _DHID)
    zpad = jnp.zeros((_DHID, _DOUT), jnp.float32)
    w2blk = jnp.concatenate(
        [jnp.concatenate([W2, zpad], axis=1),
         jnp.concatenate([zpad, W2], axis=1)], axis=0)
    g2p = pl.pallas_call(
        _mm2_body,
        out_shape=jax.ShapeDtypeStruct((_N // 2, 2 * _DOUT), jnp.float32),
    )(s1.reshape(_NC, _NT // 2, 2 * _DHID), g1p, degp2, b1pk, w2blk)

    s2 = _make_scatter(_DOUT, 20)(
        row_p, col_p, g2p.reshape(_N, _DOUT), zeros8)

    out = pl.pallas_call(
        _tail_body,
        out_shape=jax.ShapeDtypeStruct((_G, _DOUT), jnp.float32),
    )(s2.reshape(_NC, _NT // 16, 16 * _DOUT), g2p.reshape(_N // 16, 16 * _DOUT),
      degp.reshape(_NC, _NT // 16, 16), jnp.tile(b2, 16).reshape(1, 16 * _DOUT),
      batch_index.astype(jnp.int32).reshape(_N // 16, 16))
    return out


# speedup vs baseline: 1.0038x; 1.0038x over previous
import jax
import jax.numpy as jnp
from jax import lax
from jax.experimental import pallas as pl
from jax.experimental.pallas import tpu as pltpu
from jax.experimental.pallas import tpu_sc as plsc

_N = 10000
_E = 320000
_G = 64
_DIN = 128
_DHID = 64
_DOUT = 8

_NC = 2
_NS = 16
_CHUNK = 128
_NT = 10240
_RPT = _NT // _NS

_TRIPS = 80
_EPAD = _TRIPS * _CHUNK * _NC * _NS
_NCHUNKS = _EPAD // _CHUNK


def _sc_mesh():
    return plsc.VectorSubcoreMesh(core_axis_name="c", subcore_axis_name="s")


def _deg_kernel(col_hbm, out_hbm, col2d, ones_v, zb, acc, sem):
    c = lax.axis_index("c")
    s = lax.axis_index("s")
    for i in range(_CHUNK // 16):
        ones_v[pl.ds(16 * i, 16)] = jnp.ones((16,), jnp.float32)
        zb[pl.ds(16 * i, 16)] = jnp.zeros((16,), jnp.float32)
    for i in range(_RPT // _CHUNK):
        pltpu.sync_copy(zb, acc.at[pl.ds(s * _RPT + _CHUNK * i, _CHUNK)])
    trip0 = (c * _NS + s) * _TRIPS
    pltpu.sync_copy(col_hbm.at[pl.ds(trip0, _TRIPS)], col2d)
    plsc.subcore_barrier()

    @pl.loop(0, _TRIPS // 16)
    def _(g):
        descs = []
        for b in range(16):
            t = g * 16 + b
            descs.append(pltpu.async_copy(ones_v, acc.at[col2d.at[t]], sem, add=True))
        for d in descs:
            d.wait()

    plsc.subcore_barrier()
    pltpu.sync_copy(acc.at[pl.ds(s * _RPT, _RPT)], out_hbm.at[c, pl.ds(s * _RPT, _RPT)])


def _make_deg():
    return pl.kernel(
        _deg_kernel,
        out_type=jax.ShapeDtypeStruct((_NC, _NT), jnp.float32),
        mesh=_sc_mesh(),
        compiler_params=pltpu.CompilerParams(use_tc_tiling_on_sc=False),
        scratch_types=[
            pltpu.VMEM((_TRIPS, _CHUNK), jnp.int32),
            pltpu.VMEM((_CHUNK,), jnp.float32),
            pltpu.VMEM((_CHUNK,), jnp.float32),
            pltpu.MemorySpace.VMEM_SHARED((_NT,), jnp.float32),
            pltpu.SemaphoreType.DMA,
        ],
    )


def _make_scatter(d, kd):
    groups = _TRIPS // kd
    inline_zero = d % 16 == 0

    def inner(row_hbm, col_hbm, g_hbm, zeros_hbm, out_hbm,
              row2d, col2d, rows, acc, gsem, ssem):
        c = lax.axis_index("c")
        s = lax.axis_index("s")
        if inline_zero:
            r0 = rows.at[0]

            @pl.loop(0, _CHUNK)
            def _(j):
                for i in range(d // 16):
                    r0[j, pl.ds(16 * i, 16)] = jnp.zeros((16,), jnp.float32)

            for i in range(_RPT // _CHUNK):
                pltpu.sync_copy(r0, acc.at[pl.ds(s * _RPT + _CHUNK * i, _CHUNK)])
        else:
            pltpu.sync_copy(zeros_hbm.at[pl.ds(s * _RPT, _RPT)],
                            acc.at[pl.ds(s * _RPT, _RPT)])
        trip0 = (c * _NS + s) * _TRIPS
        pltpu.sync_copy(row_hbm.at[pl.ds(trip0, _TRIPS)], row2d)
        pltpu.sync_copy(col_hbm.at[pl.ds(trip0, _TRIPS)], col2d)
        plsc.subcore_barrier()

        for b in range(kd):
            pltpu.async_copy(g_hbm.at[row2d.at[b]], rows.at[b], gsem)

        @pl.loop(0, groups)
        def _(g):
            t0 = g * kd
            sdescs = []
            for b in range(kd):
                t = t0 + b
                pltpu.make_async_copy(g_hbm.at[row2d.at[t]], rows.at[b], gsem).wait()
                sdescs.append(
                    pltpu.async_copy(rows.at[b], acc.at[col2d.at[t]], ssem, add=True))
            for b in range(kd):
                sdescs[b].wait()
                tn = t0 + kd + b
                tn = jnp.where(tn >= _TRIPS, tn - _TRIPS, tn)
                pltpu.async_copy(g_hbm.at[row2d.at[tn]], rows.at[b], gsem)

        for b in range(kd):
            pltpu.make_async_copy(g_hbm.at[row2d.at[b]], rows.at[b], gsem).wait()
        plsc.subcore_barrier()
        pltpu.sync_copy(acc.at[pl.ds(s * _RPT, _RPT)],
                        out_hbm.at[c, pl.ds(s * _RPT, _RPT)])

    if inline_zero:
        def body(row_hbm, col_hbm, g_hbm, out_hbm,
                 row2d, col2d, rows, acc, gsem, ssem):
            inner(row_hbm, col_hbm, g_hbm, None, out_hbm,
                  row2d, col2d, rows, acc, gsem, ssem)
    else:
        body = inner

    return pl.kernel(
        body,
        out_type=jax.ShapeDtypeStruct((_NC, _NT, d), jnp.float32),
        mesh=_sc_mesh(),
        compiler_params=pltpu.CompilerParams(use_tc_tiling_on_sc=False),
        scratch_types=[
            pltpu.VMEM((_TRIPS, _CHUNK), jnp.int32),
            pltpu.VMEM((_TRIPS, _CHUNK), jnp.int32),
            pltpu.VMEM((kd, _CHUNK, d), jnp.float32),
            pltpu.MemorySpace.VMEM_SHARED((_NT, d), jnp.float32),
            pltpu.SemaphoreType.DMA,
            pltpu.SemaphoreType.DMA,
        ],
    )


def _sel(pairs, width):
    seg = width // pairs
    lane = lax.broadcasted_iota(jnp.int32, (pairs, width), 1)
    row = lax.broadcasted_iota(jnp.int32, (pairs, width), 0)
    return (lane // seg == row).astype(jnp.float32)


def _dis2(degp2_ref):
    d2 = degp2_ref[0] + degp2_ref[1]
    return lax.rsqrt(d2[:_N // 2] + 1.0)


def _mm1_body(xpk_ref, w1blk_ref, degp2_ref, g1p_ref):
    dis_pk = lax.dot_general(_dis2(degp2_ref), _sel(2, 128), (((1,), (0,)), ((), ())),
                             preferred_element_type=jnp.float32)
    hpk = jnp.dot(xpk_ref[...], w1blk_ref[...],
                  preferred_element_type=jnp.float32)
    g1p_ref[...] = hpk * dis_pk


def _mm2_body(s1p_ref, g1p_ref, degp2_ref, b1pk_ref, w2blk_ref, g2p_ref):
    cn = (((1,), (0,)), ((), ()))
    dis2 = _dis2(degp2_ref)
    dis_pk = lax.dot_general(dis2, _sel(2, 128), cn,
                             preferred_element_type=jnp.float32)
    agg = (s1p_ref[0] + s1p_ref[1])[:_N // 2] + g1p_ref[...]
    h1pk = jnp.tanh(dis_pk * agg + b1pk_ref[...])
    z2pk = jnp.dot(h1pk, w2blk_ref[...],
                   preferred_element_type=jnp.float32)
    dis_pk16 = lax.dot_general(dis2, _sel(2, 16), cn,
                               preferred_element_type=jnp.float32)
    g2p_ref[...] = z2pk * dis_pk16


def _tail_body(s2p_ref, g2p16_ref, degp16_ref, b2pk_ref, batchp_ref, out_ref):
    cn = (((1,), (0,)), ((), ()))
    d16 = degp16_ref[0] + degp16_ref[1]
    dis16 = lax.rsqrt(d16[:_N // 16] + 1.0)
    dis_pk = lax.dot_general(dis16, _sel(16, 128), cn,
                             preferred_element_type=jnp.float32)
    agg = (s2p_ref[0] + s2p_ref[1])[:_N // 16] + g2p16_ref[...]
    h2pk = jnp.tanh(dis_pk * agg + b2pk_ref[...])
    batchp = batchp_ref[...]
    gid = lax.broadcasted_iota(jnp.int32, (1, _G), 1)
    ones = jnp.ones((_N // 16, 1), jnp.float32)
    dn0 = (((0,), (0,)), ((), ()))
    sums = jnp.zeros((_G, _DOUT), jnp.float32)
    cnt = jnp.zeros((_G, 1), jnp.float32)
    for k in range(16):
        mk = (batchp[:, k:k + 1] == gid).astype(jnp.float32)
        hk = h2pk[:, 8 * k:8 * k + 8]
        sums = sums + lax.dot_general(mk, hk, dn0,
                                      preferred_element_type=jnp.float32)
        cnt = cnt + lax.dot_general(mk, ones, dn0,
                                    preferred_element_type=jnp.float32)
    mean = sums / jnp.maximum(cnt, 1.0)
    out_ref[...] = 1.0 / (1.0 + jnp.exp(-mean))


def kernel(x, edge_index, batch_index, W1, b1, W2, b2):
    row = edge_index[0].astype(jnp.int32)
    col = edge_index[1].astype(jnp.int32)
    npad = _EPAD - _E
    pad_r = (jnp.arange(npad, dtype=jnp.int32) * 37) % _N
    pad_c = _N + (jnp.arange(npad, dtype=jnp.int32) % (_NT - _N))
    row_p = jnp.concatenate([row, pad_r]).reshape(_NCHUNKS, _CHUNK)
    col_p = jnp.concatenate([col, pad_c]).reshape(_NCHUNKS, _CHUNK)

    zeros8 = jnp.zeros((_NT, _DOUT), jnp.float32)

    degp = _make_deg()(col_p)

    degp2 = degp.reshape(_NC, _NT // 2, 2)
    zpad1 = jnp.zeros((_DIN, _DHID), jnp.float32)
    w1blk = jnp.concatenate(
        [jnp.concatenate([W1, zpad1], axis=1),
         jnp.concatenate([zpad1, W1], axis=1)], axis=0)
    g1p = pl.pallas_call(
        _mm1_body,
        out_shape=jax.ShapeDtypeStruct((_N // 2, 2 * _DHID), jnp.float32),
    )(x.reshape(_N // 2, 2 * _DIN), w1blk, degp2)

    s1 = _make_scatter(_DHID, 8)(
        row_p, col_p, g1p.reshape(_N, _DHID))

    b1pk = jnp.concatenate([b1, b1]).reshape(1, 2 * _DHID)
    zpad = jnp.zeros((_DHID, _DOUT), jnp.float32)
    w2blk = jnp.concatenate(
        [jnp.concatenate([W2, zpad], axis=1),
         jnp.concatenate([zpad, W2], axis=1)], axis=0)
    g2p = pl.pallas_call(
        _mm2_body,
        out_shape=jax.ShapeDtypeStruct((_N // 2, 2 * _DOUT), jnp.float32),
    )(s1.reshape(_NC, _NT // 2, 2 * _DHID), g1p, degp2, b1pk, w2blk)

    s2 = _make_scatter(_DOUT, 16)(
        row_p, col_p, g2p.reshape(_N, _DOUT), zeros8)

    out = pl.pallas_call(
        _tail_body,
        out_shape=jax.ShapeDtypeStruct((_G, _DOUT), jnp.float32),
    )(s2.reshape(_NC, _NT // 16, 16 * _DOUT), g2p.reshape(_N // 16, 16 * _DOUT),
      degp.reshape(_NC, _NT // 16, 16), jnp.tile(b2, 16).reshape(1, 16 * _DOUT),
      batch_index.astype(jnp.int32).reshape(_N // 16, 16))
    return out


# speedup vs baseline: 1.0046x; 1.0008x over previous
import jax
import jax.numpy as jnp
from jax import lax
from jax.experimental import pallas as pl
from jax.experimental.pallas import tpu as pltpu
from jax.experimental.pallas import tpu_sc as plsc

_N = 10000
_E = 320000
_G = 64
_DIN = 128
_DHID = 64
_DOUT = 8

_NC = 2
_NS = 16
_CHUNK = 128
_NT = 10240
_RPT = _NT // _NS

_TRIPS = 80
_EPAD = _TRIPS * _CHUNK * _NC * _NS
_NCHUNKS = _EPAD // _CHUNK


def _sc_mesh():
    return plsc.VectorSubcoreMesh(core_axis_name="c", subcore_axis_name="s")


def _deg_kernel(col_hbm, out_hbm, col2d, ones_v, zb, acc, sem):
    c = lax.axis_index("c")
    s = lax.axis_index("s")
    for i in range(_CHUNK // 16):
        ones_v[pl.ds(16 * i, 16)] = jnp.ones((16,), jnp.float32)
        zb[pl.ds(16 * i, 16)] = jnp.zeros((16,), jnp.float32)
    for i in range(_RPT // _CHUNK):
        pltpu.sync_copy(zb, acc.at[pl.ds(s * _RPT + _CHUNK * i, _CHUNK)])
    trip0 = (c * _NS + s) * _TRIPS
    pltpu.sync_copy(col_hbm.at[pl.ds(trip0, _TRIPS)], col2d)
    plsc.subcore_barrier()

    @pl.loop(0, _TRIPS // 16)
    def _(g):
        descs = []
        for b in range(16):
            t = g * 16 + b
            descs.append(pltpu.async_copy(ones_v, acc.at[col2d.at[t]], sem, add=True))
        for d in descs:
            d.wait()

    plsc.subcore_barrier()
    pltpu.sync_copy(acc.at[pl.ds(s * _RPT, _RPT)], out_hbm.at[c, pl.ds(s * _RPT, _RPT)])


def _make_deg():
    return pl.kernel(
        _deg_kernel,
        out_type=jax.ShapeDtypeStruct((_NC, _NT), jnp.float32),
        mesh=_sc_mesh(),
        compiler_params=pltpu.CompilerParams(use_tc_tiling_on_sc=False),
        scratch_types=[
            pltpu.VMEM((_TRIPS, _CHUNK), jnp.int32),
            pltpu.VMEM((_CHUNK,), jnp.float32),
            pltpu.VMEM((_CHUNK,), jnp.float32),
            pltpu.MemorySpace.VMEM_SHARED((_NT,), jnp.float32),
            pltpu.SemaphoreType.DMA,
        ],
    )


def _make_scatter(d, kd):
    groups = _TRIPS // kd
    inline_zero = d % 16 == 0

    def inner(row_hbm, col_hbm, g_hbm, zeros_hbm, out_hbm,
              row2d, col2d, rows, acc, gsem, ssem):
        c = lax.axis_index("c")
        s = lax.axis_index("s")
        if inline_zero:
            r0 = rows.at[0]

            @pl.loop(0, _CHUNK)
            def _(j):
                for i in range(d // 16):
                    r0[j, pl.ds(16 * i, 16)] = jnp.zeros((16,), jnp.float32)

            for i in range(_RPT // _CHUNK):
                pltpu.sync_copy(r0, acc.at[pl.ds(s * _RPT + _CHUNK * i, _CHUNK)])
        else:
            pltpu.sync_copy(zeros_hbm.at[pl.ds(s * _RPT, _RPT)],
                            acc.at[pl.ds(s * _RPT, _RPT)])
        trip0 = (c * _NS + s) * _TRIPS
        pltpu.sync_copy(row_hbm.at[pl.ds(trip0, _TRIPS)], row2d)
        pltpu.sync_copy(col_hbm.at[pl.ds(trip0, _TRIPS)], col2d)
        for b in range(kd):
            pltpu.async_copy(g_hbm.at[row2d.at[b]], rows.at[b], gsem)
        plsc.subcore_barrier()

        @pl.loop(0, groups)
        def _(g):
            t0 = g * kd
            sdescs = []
            for b in range(kd):
                t = t0 + b
                pltpu.make_async_copy(g_hbm.at[row2d.at[t]], rows.at[b], gsem).wait()
                sdescs.append(
                    pltpu.async_copy(rows.at[b], acc.at[col2d.at[t]], ssem, add=True))
            for b in range(kd):
                sdescs[b].wait()
                tn = t0 + kd + b
                tn = jnp.where(tn >= _TRIPS, tn - _TRIPS, tn)
                pltpu.async_copy(g_hbm.at[row2d.at[tn]], rows.at[b], gsem)

        for b in range(kd):
            pltpu.make_async_copy(g_hbm.at[row2d.at[b]], rows.at[b], gsem).wait()
        plsc.subcore_barrier()
        pltpu.sync_copy(acc.at[pl.ds(s * _RPT, _RPT)],
                        out_hbm.at[c, pl.ds(s * _RPT, _RPT)])

    if inline_zero:
        def body(row_hbm, col_hbm, g_hbm, out_hbm,
                 row2d, col2d, rows, acc, gsem, ssem):
            inner(row_hbm, col_hbm, g_hbm, None, out_hbm,
                  row2d, col2d, rows, acc, gsem, ssem)
    else:
        body = inner

    return pl.kernel(
        body,
        out_type=jax.ShapeDtypeStruct((_NC, _NT, d), jnp.float32),
        mesh=_sc_mesh(),
        compiler_params=pltpu.CompilerParams(use_tc_tiling_on_sc=False),
        scratch_types=[
            pltpu.VMEM((_TRIPS, _CHUNK), jnp.int32),
            pltpu.VMEM((_TRIPS, _CHUNK), jnp.int32),
            pltpu.VMEM((kd, _CHUNK, d), jnp.float32),
            pltpu.MemorySpace.VMEM_SHARED((_NT, d), jnp.float32),
            pltpu.SemaphoreType.DMA,
            pltpu.SemaphoreType.DMA,
        ],
    )


def _sel(pairs, width):
    seg = width // pairs
    lane = lax.broadcasted_iota(jnp.int32, (pairs, width), 1)
    row = lax.broadcasted_iota(jnp.int32, (pairs, width), 0)
    return (lane // seg == row).astype(jnp.float32)


def _dis2(degp2_ref):
    d2 = degp2_ref[0] + degp2_ref[1]
    return lax.rsqrt(d2[:_N // 2] + 1.0)


def _mm1_body(xpk_ref, w1blk_ref, degp2_ref, g1p_ref):
    dis_pk = lax.dot_general(_dis2(degp2_ref), _sel(2, 128), (((1,), (0,)), ((), ())),
                             preferred_element_type=jnp.float32)
    hpk = jnp.dot(xpk_ref[...], w1blk_ref[...],
                  preferred_element_type=jnp.float32)
    g1p_ref[...] = hpk * dis_pk


def _mm2_body(s1p_ref, g1p_ref, degp2_ref, b1pk_ref, w2blk_ref, g2p_ref):
    cn = (((1,), (0,)), ((), ()))
    dis2 = _dis2(degp2_ref)
    dis_pk = lax.dot_general(dis2, _sel(2, 128), cn,
                             preferred_element_type=jnp.float32)
    agg = (s1p_ref[0] + s1p_ref[1])[:_N // 2] + g1p_ref[...]
    h1pk = jnp.tanh(dis_pk * agg + b1pk_ref[...])
    z2pk = jnp.dot(h1pk, w2blk_ref[...],
                   preferred_element_type=jnp.float32)
    dis_pk16 = lax.dot_general(dis2, _sel(2, 16), cn,
                               preferred_element_type=jnp.float32)
    g2p_ref[...] = z2pk * dis_pk16


def _tail_body(s2p_ref, g2p16_ref, degp16_ref, b2pk_ref, batchp_ref, out_ref):
    cn = (((1,), (0,)), ((), ()))
    d16 = degp16_ref[0] + degp16_ref[1]
    dis16 = lax.rsqrt(d16[:_N // 16] + 1.0)
    dis_pk = lax.dot_general(dis16, _sel(16, 128), cn,
                             preferred_element_type=jnp.float32)
    agg = (s2p_ref[0] + s2p_ref[1])[:_N // 16] + g2p16_ref[...]
    h2pk = jnp.tanh(dis_pk * agg + b2pk_ref[...])
    batchp = batchp_ref[...]
    gid = lax.broadcasted_iota(jnp.int32, (1, _G), 1)
    ones = jnp.ones((_N // 16, 1), jnp.float32)
    dn0 = (((0,), (0,)), ((), ()))
    sums = jnp.zeros((_G, _DOUT), jnp.float32)
    cnt = jnp.zeros((_G, 1), jnp.float32)
    for k in range(16):
        mk = (batchp[:, k:k + 1] == gid).astype(jnp.float32)
        hk = h2pk[:, 8 * k:8 * k + 8]
        sums = sums + lax.dot_general(mk, hk, dn0,
                                      preferred_element_type=jnp.float32)
        cnt = cnt + lax.dot_general(mk, ones, dn0,
                                    preferred_element_type=jnp.float32)
    mean = sums / jnp.maximum(cnt, 1.0)
    out_ref[...] = 1.0 / (1.0 + jnp.exp(-mean))


def kernel(x, edge_index, batch_index, W1, b1, W2, b2):
    row = edge_index[0].astype(jnp.int32)
    col = edge_index[1].astype(jnp.int32)
    npad = _EPAD - _E
    pad_r = (jnp.arange(npad, dtype=jnp.int32) * 37) % _N
    pad_c = _N + (jnp.arange(npad, dtype=jnp.int32) % (_NT - _N))
    row_p = jnp.concatenate([row, pad_r]).reshape(_NCHUNKS, _CHUNK)
    col_p = jnp.concatenate([col, pad_c]).reshape(_NCHUNKS, _CHUNK)

    zeros8 = jnp.zeros((_NT, _DOUT), jnp.float32)

    degp = _make_deg()(col_p)

    degp2 = degp.reshape(_NC, _NT // 2, 2)
    zpad1 = jnp.zeros((_DIN, _DHID), jnp.float32)
    w1blk = jnp.concatenate(
        [jnp.concatenate([W1, zpad1], axis=1),
         jnp.concatenate([zpad1, W1], axis=1)], axis=0)
    g1p = pl.pallas_call(
        _mm1_body,
        out_shape=jax.ShapeDtypeStruct((_N // 2, 2 * _DHID), jnp.float32),
    )(x.reshape(_N // 2, 2 * _DIN), w1blk, degp2)

    s1 = _make_scatter(_DHID, 8)(
        row_p, col_p, g1p.reshape(_N, _DHID))

    b1pk = jnp.concatenate([b1, b1]).reshape(1, 2 * _DHID)
    zpad = jnp.zeros((_DHID, _DOUT), jnp.float32)
    w2blk = jnp.concatenate(
        [jnp.concatenate([W2, zpad], axis=1),
         jnp.concatenate([zpad, W2], axis=1)], axis=0)
    g2p = pl.pallas_call(
        _mm2_body,
        out_shape=jax.ShapeDtypeStruct((_N // 2, 2 * _DOUT), jnp.float32),
    )(s1.reshape(_NC, _NT // 2, 2 * _DHID), g1p, degp2, b1pk, w2blk)

    s2 = _make_scatter(_DOUT, 16)(
        row_p, col_p, g2p.reshape(_N, _DOUT), zeros8)

    out = pl.pallas_call(
        _tail_body,
        out_shape=jax.ShapeDtypeStruct((_G, _DOUT), jnp.float32),
    )(s2.reshape(_NC, _NT // 16, 16 * _DOUT), g2p.reshape(_N // 16, 16 * _DOUT),
      degp.reshape(_NC, _NT // 16, 16), jnp.tile(b2, 16).reshape(1, 16 * _DOUT),
      batch_index.astype(jnp.int32).reshape(_N // 16, 16))
    return out
